# baseline (speedup 1.0000x reference)
import dataclasses

import jax
import jax.numpy as jnp
from jax import lax
from jax.experimental import pallas as pl
from jax.experimental.pallas import tpu as pltpu
from jax.experimental.pallas import tpu_sc as plsc

B = 16384
F = 26
K = 16
VOCAB = 100000
D_F = 13

NC, NS = 2, 16
NW = NC * NS
BPW = B // NW
IPW = BPW * F
GROUP = 64
G = GROUP * F
NG = BPW // GROUP


def _sc_body(xflat_hbm, embl_hbm, emb2_hbm, acc_hbm, sq_hbm, lin_hbm,
             idx_v, lin_vals, rb0, rb1, acc_v, sq_v, lin_v,
             lin_sem, sem0, sem1):
    wid = lax.axis_index("s") * NC + lax.axis_index("c")
    base = wid * BPW
    ibase = wid * IPW

    pltpu.sync_copy(xflat_hbm.at[pl.ds(ibase, IPW)], idx_v)

    iota = lax.iota(jnp.int32, 16)

    @pl.loop(0, IPW // 16)
    def _(r):
        s = pl.multiple_of(r * 16, 16)
        pos = iota + s
        off = lax.rem(pos, F) * VOCAB
        idx_v[pl.ds(s, 16)] = idx_v[pl.ds(s, 16)] + off

    lin_cp = pltpu.make_async_copy(embl_hbm.at[idx_v], lin_vals, lin_sem)
    lin_cp.start()

    bufs = (rb0, rb1)
    sems = (sem0, sem1)

    def gather(g):
        i = g % 2
        return pltpu.make_async_copy(
            emb2_hbm.at[idx_v.at[pl.ds(g * G, G)]], bufs[i], sems[i])

    gather(0).start()
    for g in range(NG):
        if g + 1 < NG:
            gather(g + 1).start()
        gather(g).wait()
        rb = bufs[g % 2]

        @pl.loop(0, GROUP)
        def _(j):
            r0 = j * F
            v = rb[r0]
            acc = v
            sq = v * v
            for f in range(1, F):
                v = rb[r0 + f]
                acc = acc + v
                sq = sq + v * v
            jj = g * GROUP + j
            acc_v[jj] = acc
            sq_v[jj] = sq

    lin_cp.wait()

    @pl.loop(0, BPW // 16)
    def _(t):
        b16 = iota * F + t * (16 * F)
        s = plsc.load_gather(lin_vals, [b16])
        for f in range(1, F):
            s = s + plsc.load_gather(lin_vals, [b16 + f])
        lin_v[pl.ds(pl.multiple_of(t * 16, 16), 16)] = s

    pltpu.sync_copy(acc_v, acc_hbm.at[pl.ds(base, BPW)])
    pltpu.sync_copy(sq_v, sq_hbm.at[pl.ds(base, BPW)])
    pltpu.sync_copy(lin_v, lin_hbm.at[pl.ds(base, BPW)])


_sc_params = pltpu.CompilerParams(
    needs_layout_passes=False, use_tc_tiling_on_sc=False)

_sc_gather_reduce = pl.kernel(
    _sc_body,
    compiler_params=_sc_params,
    out_type=(
        jax.ShapeDtypeStruct((B, K), jnp.float32),
        jax.ShapeDtypeStruct((B, K), jnp.float32),
        jax.ShapeDtypeStruct((B,), jnp.float32),
    ),
    mesh=plsc.VectorSubcoreMesh(core_axis_name="c", subcore_axis_name="s"),
    scratch_types=[
        pltpu.VMEM((IPW,), jnp.int32),
        pltpu.VMEM((IPW,), jnp.float32),
        pltpu.VMEM((G, K), jnp.float32),
        pltpu.VMEM((G, K), jnp.float32),
        pltpu.VMEM((BPW, K), jnp.float32),
        pltpu.VMEM((BPW, K), jnp.float32),
        pltpu.VMEM((BPW,), jnp.float32),
        pltpu.SemaphoreType.DMA,
        pltpu.SemaphoreType.DMA,
        pltpu.SemaphoreType.DMA,
    ],
)

BLK = 2048


def _combine_body(acc_ref, sq_ref, lin_ref, dx_ref, w2_ref, wl_ref, b_ref,
                  o_ref):
    dx = dx_ref[...]
    acc = acc_ref[...]
    wl = wl_ref[...]
    dvec = jnp.zeros((BLK, K), jnp.float32)
    dlin = jnp.zeros((BLK, 1), jnp.float32)
    dsq = jnp.zeros((BLK, 1), jnp.float32)
    for f in range(D_F):
        c = dx[:, f:f + 1]
        w2f = w2_ref[f:f + 1, :]
        dvec = dvec + c * w2f
        dlin = dlin + c * wl[0:1, f:f + 1]
        s2f = jnp.sum(w2f * w2f, axis=1, keepdims=True)
        dsq = dsq + (c * c) * s2f
    tot = acc + dvec
    a = jnp.sum(tot * tot, axis=1, keepdims=True)
    bterm = jnp.sum(sq_ref[...], axis=1, keepdims=True) + dsq
    o_ref[...] = b_ref[...] + lin_ref[...] + dlin + 0.5 * (a - bterm)


_combine = pl.pallas_call(
    _combine_body,
    out_shape=jax.ShapeDtypeStruct((B, 1), jnp.float32),
    grid=(B // BLK,),
    in_specs=[
        pl.BlockSpec((BLK, K), lambda i: (i, 0)),
        pl.BlockSpec((BLK, K), lambda i: (i, 0)),
        pl.BlockSpec((BLK, 1), lambda i: (i, 0)),
        pl.BlockSpec((BLK, D_F), lambda i: (i, 0)),
        pl.BlockSpec((D_F, K), lambda i: (0, 0)),
        pl.BlockSpec((1, D_F), lambda i: (0, 0)),
        pl.BlockSpec((1, 1), lambda i: (0, 0)),
    ],
    out_specs=pl.BlockSpec((BLK, 1), lambda i: (i, 0)),
)


def kernel(sparse_x, dense_x, bias, emb_linear, dense_linear_w, emb_second,
           dense_second_w):
    x_flat = sparse_x.reshape(B * F)
    embl = emb_linear.reshape(-1)
    acc, sq, lin = _sc_gather_reduce(x_flat, embl, emb_second)
    out = _combine(
        acc, sq, lin.reshape(B, 1), dense_x,
        dense_second_w.reshape(D_F, K),
        dense_linear_w.reshape(1, D_F),
        bias.reshape(1, 1),
    )
    return out.reshape(B)


# speedup vs baseline: 2.9481x; 2.9481x over previous
import jax
import jax.numpy as jnp
from jax import lax
from jax.experimental import pallas as pl
from jax.experimental.pallas import tpu as pltpu
from jax.experimental.pallas import tpu_sc as plsc

B = 16384
F = 26
K = 16
VOCAB = 100000
V = VOCAB * F
D_F = 13

NC, NS = 2, 16
NW = NC * NS
BPW = B // NW
IPW = BPW * F

CH = 32768
NCH = -(-V // CH)


def _planes_body(e2t_ref, lt_ref, *o_refs):
    for k in range(K):
        o_refs[k][...] = e2t_ref[k, :]
    o_refs[K][...] = lt_ref[0, :]


_format_planes = pl.pallas_call(
    _planes_body,
    out_shape=tuple(
        jax.ShapeDtypeStruct((V,), jnp.float32) for _ in range(K + 1)),
    grid=(NCH,),
    in_specs=[
        pl.BlockSpec((K, CH), lambda i: (0, i)),
        pl.BlockSpec((1, CH), lambda i: (0, i)),
    ],
    out_specs=tuple(pl.BlockSpec((CH,), lambda i: (i,)) for _ in range(K + 1)),
)

_sc_params = pltpu.CompilerParams(
    needs_layout_passes=False, use_tc_tiling_on_sc=False)

NPL = K + 1


def _sc_body(xflat_hbm, *refs):
    planes = refs[:NPL]
    acc_hbm, sq_hbm, lin_hbm = refs[NPL:NPL + 3]
    idx_v, val0, val1, acc_v, sq_v, lin_v = refs[NPL + 3:NPL + 9]
    sem0, sem1 = refs[NPL + 9:NPL + 11]

    wid = lax.axis_index("s") * NC + lax.axis_index("c")
    base = wid * BPW
    ibase = wid * IPW

    pltpu.sync_copy(xflat_hbm.at[pl.ds(ibase, IPW)], idx_v)

    iota = lax.iota(jnp.int32, 16)

    @pl.loop(0, IPW // 16)
    def _(r):
        s = pl.multiple_of(r * 16, 16)
        pos = iota + s
        off = lax.rem(pos, F) * VOCAB
        idx_v[pl.ds(s, 16)] = idx_v[pl.ds(s, 16)] + off

    bufs = (val0, val1)
    sems = (sem0, sem1)

    def gather(p):
        i = p % 2
        return pltpu.make_async_copy(planes[p].at[idx_v], bufs[i], sems[i])

    def reduce_plane(p):
        vals = bufs[p % 2]

        @pl.loop(0, BPW // 16)
        def _(t):
            b16 = iota * F + t * (16 * F)
            s = plsc.load_gather(vals, [b16])
            q = s * s
            for f in range(1, F):
                v = plsc.load_gather(vals, [b16 + f])
                s = s + v
                q = q + v * v
            col = pl.multiple_of(t * 16, 16)
            if p < K:
                acc_v[p, pl.ds(col, 16)] = s
                sq_v[p, pl.ds(col, 16)] = q
            else:
                lin_v[pl.ds(col, 16)] = s

    gather(0).start()
    for p in range(NPL):
        if p + 1 < NPL:
            gather(p + 1).start()
        gather(p).wait()
        reduce_plane(p)

    pltpu.sync_copy(acc_v, acc_hbm.at[:, pl.ds(base, BPW)])
    pltpu.sync_copy(sq_v, sq_hbm.at[:, pl.ds(base, BPW)])
    pltpu.sync_copy(lin_v, lin_hbm.at[0, pl.ds(base, BPW)])


_sc_gather_reduce = pl.kernel(
    _sc_body,
    compiler_params=_sc_params,
    out_type=(
        jax.ShapeDtypeStruct((K, B), jnp.float32),
        jax.ShapeDtypeStruct((K, B), jnp.float32),
        jax.ShapeDtypeStruct((1, B), jnp.float32),
    ),
    mesh=plsc.VectorSubcoreMesh(core_axis_name="c", subcore_axis_name="s"),
    scratch_types=[
        pltpu.VMEM((IPW,), jnp.int32),
        pltpu.VMEM((IPW,), jnp.float32),
        pltpu.VMEM((IPW,), jnp.float32),
        pltpu.VMEM((K, BPW), jnp.float32),
        pltpu.VMEM((K, BPW), jnp.float32),
        pltpu.VMEM((BPW,), jnp.float32),
        pltpu.SemaphoreType.DMA,
        pltpu.SemaphoreType.DMA,
    ],
)

BLK = 2048


def _combine_body(acc_ref, sq_ref, lin_ref, dxt_ref, w2t_ref, wl_ref, b_ref,
                  o_ref):
    dxt = dxt_ref[...]
    accT = acc_ref[...]
    w2t = w2t_ref[...]
    wl = wl_ref[...]
    dvecT = jnp.zeros_like(accT)
    dlinT = jnp.zeros((1, BLK), jnp.float32)
    dsqT = jnp.zeros((1, BLK), jnp.float32)
    for f in range(D_F):
        row = dxt[f:f + 1, :]
        col = w2t[:, f:f + 1]
        dvecT = dvecT + col * row
        dlinT = dlinT + wl[0:1, f:f + 1] * row
        s2f = jnp.sum(col * col, axis=0, keepdims=True)
        dsqT = dsqT + s2f * (row * row)
    totT = accT + dvecT
    aT = jnp.sum(totT * totT, axis=0, keepdims=True)
    bT = jnp.sum(sq_ref[...], axis=0, keepdims=True) + dsqT
    o_ref[...] = b_ref[...] + lin_ref[...] + dlinT + 0.5 * (aT - bT)


_combine = pl.pallas_call(
    _combine_body,
    out_shape=jax.ShapeDtypeStruct((1, B), jnp.float32),
    grid=(B // BLK,),
    in_specs=[
        pl.BlockSpec((K, BLK), lambda i: (0, i)),
        pl.BlockSpec((K, BLK), lambda i: (0, i)),
        pl.BlockSpec((1, BLK), lambda i: (0, i)),
        pl.BlockSpec((D_F, BLK), lambda i: (0, i)),
        pl.BlockSpec((K, D_F), lambda i: (0, 0)),
        pl.BlockSpec((1, D_F), lambda i: (0, 0)),
        pl.BlockSpec((1, 1), lambda i: (0, 0)),
    ],
    out_specs=pl.BlockSpec((1, BLK), lambda i: (0, i)),
)


def kernel(sparse_x, dense_x, bias, emb_linear, dense_linear_w, emb_second,
           dense_second_w):
    x_flat = sparse_x.reshape(B * F)
    planes = _format_planes(emb_second.T, emb_linear.T)
    accT, sqT, lin = _sc_gather_reduce(x_flat, *planes)
    out = _combine(
        accT, sqT, lin, dense_x.T,
        dense_second_w.reshape(D_F, K).T,
        dense_linear_w.reshape(1, D_F),
        bias.reshape(1, 1),
    )
    return out.reshape(B)
